# baseline (speedup 1.0000x reference)
import jax
import jax.numpy as jnp
from jax.experimental import pallas as pl
from jax.experimental.pallas import tpu as pltpu

_CB = 4
_D = 6


def _softmax_t_kernel(a_ref, att_ref):
    a = a_ref[...]
    m = jnp.max(a, axis=1, keepdims=True)
    e = jnp.exp(a - m)
    att_ref[...] = (e / jnp.sum(e, axis=1, keepdims=True)).T


def _mm_kernel(att_ref, x_ref, o_ref, ibuf, obuf, insem, outsem):
    nb, c, l, v = x_ref.shape
    ncb = c // _CB
    attT = att_ref[...]

    def in_copy(i, slot):
        return pltpu.make_async_copy(
            x_ref.at[:, pl.ds(i * _CB, _CB)], ibuf.at[slot], insem.at[slot])

    def out_copy_one(i, slot, cc):
        return pltpu.make_async_copy(
            obuf.at[slot, cc],
            o_ref.at[i * _CB + cc],
            outsem.at[slot])

    for k in range(_D):
        in_copy(k, k).start()

    def body(i, carry):
        slot = jax.lax.rem(i, _D)
        in_copy(i, slot).wait()

        @pl.when(i >= _D)
        def _():
            for cc in range(_CB):
                out_copy_one(i - _D, slot, cc).wait()

        for cc in range(_CB):
            for nn in range(nb):
                obuf[slot, cc, nn] = jnp.dot(
                    ibuf[slot, nn, cc], attT,
                    preferred_element_type=jnp.float32)

        for cc in range(_CB):
            out_copy_one(i, slot, cc).start()

        @pl.when(i + _D < ncb)
        def _():
            in_copy(i + _D, slot).start()

        return carry

    jax.lax.fori_loop(0, ncb, body, 0)

    for k in range(_D):
        i = ncb - _D + k
        for cc in range(_CB):
            out_copy_one(i, i % _D, cc).wait()


def kernel(x, edge_list):
    n, c, v, l = x.shape
    xT = jnp.swapaxes(x, 2, 3)

    attT = pl.pallas_call(
        _softmax_t_kernel,
        out_shape=jax.ShapeDtypeStruct((v, v), jnp.float32),
    )(edge_list)

    ot = pl.pallas_call(
        _mm_kernel,
        in_specs=[
            pl.BlockSpec(memory_space=pltpu.MemorySpace.VMEM),
            pl.BlockSpec(memory_space=pltpu.MemorySpace.HBM),
        ],
        out_specs=pl.BlockSpec(memory_space=pltpu.MemorySpace.HBM),
        out_shape=jax.ShapeDtypeStruct((c, n, l, v), jnp.float32),
        scratch_shapes=[
            pltpu.VMEM((_D, n, _CB, l, v), jnp.float32),
            pltpu.VMEM((_D, _CB, n, l, v), jnp.float32),
            pltpu.SemaphoreType.DMA((_D,)),
            pltpu.SemaphoreType.DMA((_D,)),
        ],
    )(attT, xT)
    return jnp.transpose(ot, (0, 1, 3, 2)).reshape(c, n * v, l)


# speedup vs baseline: 1.0027x; 1.0027x over previous
import jax
import jax.numpy as jnp
from jax.experimental import pallas as pl
from jax.experimental.pallas import tpu as pltpu

_CB = 4
_D = 6


def _softmax_t_kernel(a_ref, att_ref):
    a = a_ref[...]
    m = jnp.max(a, axis=1, keepdims=True)
    e = jnp.exp(a - m)
    att_ref[...] = (e / jnp.sum(e, axis=1, keepdims=True)).T


def _mm_kernel(att_ref, x_ref, o_ref, ibuf, obuf, insem, outsem):
    nb, c, l, v = x_ref.shape
    ncb = c // _CB
    attT = att_ref[...]

    def in_copy(i, slot):
        return pltpu.make_async_copy(
            x_ref.at[:, pl.ds(i * _CB, _CB)], ibuf.at[slot], insem.at[slot])

    def out_copy_one(i, slot, cc):
        return pltpu.make_async_copy(
            obuf.at[slot, cc],
            o_ref.at[i * _CB + cc],
            outsem.at[slot])

    for k in range(_D):
        in_copy(k, k).start()

    def body(i, carry):
        slot = jax.lax.rem(i, _D)
        in_copy(i, slot).wait()

        @pl.when(i >= _D)
        def _():
            for cc in range(_CB):
                out_copy_one(i - _D, slot, cc).wait()

        for cc in range(_CB):
            for nn in range(nb):
                obuf[slot, cc, nn] = jnp.dot(
                    ibuf[slot, nn, cc], attT,
                    preferred_element_type=jnp.float32)

        for cc in range(_CB):
            out_copy_one(i, slot, cc).start()

        @pl.when(i + _D < ncb)
        def _():
            in_copy(i + _D, slot).start()

        return carry

    jax.lax.fori_loop(0, ncb, body, 0)

    for k in range(_D):
        i = ncb - _D + k
        for cc in range(_CB):
            out_copy_one(i, i % _D, cc).wait()


def kernel(x, edge_list):
    n, c, v, l = x.shape
    xT = jnp.swapaxes(x, 2, 3)

    attT = pl.pallas_call(
        _softmax_t_kernel,
        out_shape=jax.ShapeDtypeStruct((v, v), jnp.float32),
    )(edge_list)

    ot = pl.pallas_call(
        _mm_kernel,
        in_specs=[
            pl.BlockSpec(memory_space=pltpu.MemorySpace.VMEM),
            pl.BlockSpec(memory_space=pltpu.MemorySpace.HBM),
        ],
        out_specs=pl.BlockSpec(memory_space=pltpu.MemorySpace.HBM),
        out_shape=jax.ShapeDtypeStruct((c, n, l, v), jnp.float32),
        compiler_params=pltpu.CompilerParams(
            flags={"XLA_SET_SPLIT_INPUT_OUTPUT_DMAS": True},
        ),
        scratch_shapes=[
            pltpu.VMEM((_D, n, _CB, l, v), jnp.float32),
            pltpu.VMEM((_D, _CB, n, l, v), jnp.float32),
            pltpu.SemaphoreType.DMA((_D,)),
            pltpu.SemaphoreType.DMA((_D,)),
        ],
    )(attT, xT)
    return jnp.transpose(ot, (0, 1, 3, 2)).reshape(c, n * v, l)
